# Initial kernel scaffold; baseline (speedup 1.0000x reference)
import jax
import jax.numpy as jnp
from jax.experimental import pallas as pl


def kernel(x, table):
    raise NotImplementedError("write your pallas kernel here")


# speedup vs baseline: 1.2184x; 1.2184x over previous
import functools

import jax
import jax.numpy as jnp
from jax import lax
from jax.experimental import pallas as pl
from jax.experimental.pallas import tpu as pltpu
from jax.experimental.pallas import tpu_sc as plsc

NC, NS = 2, 16
NW = NC * NS


@functools.partial(jax.jit, static_argnums=(2, 3))
def _sc_gather(idx_flat, table, b_per_w, chunk):
    n_chunks = b_per_w // chunk
    D = table.shape[1]
    B = b_per_w * NW
    mesh = plsc.VectorSubcoreMesh(core_axis_name="c", subcore_axis_name="s")

    @functools.partial(
        pl.kernel,
        out_type=jax.ShapeDtypeStruct((B, D), jnp.float32),
        mesh=mesh,
        scratch_types=[
            pltpu.VMEM((b_per_w,), jnp.int32),
            pltpu.VMEM((2, chunk, D), jnp.float32),
            pltpu.SemaphoreType.DMA,
            pltpu.SemaphoreType.DMA,
        ],
        compiler_params=pltpu.CompilerParams(use_tc_tiling_on_sc=False),
    )
    def k(idx_hbm, table_hbm, out_hbm, idx_v, rows_v, gsem, osem):
        wid = lax.axis_index("s") * NC + lax.axis_index("c")
        base = wid * b_per_w
        pltpu.sync_copy(idx_hbm.at[pl.ds(base, b_per_w)], idx_v)

        def gstart(i):
            c = pltpu.make_async_copy(
                table_hbm.at[idx_v.at[pl.ds(i * chunk, chunk)]],
                rows_v.at[i % 2],
                gsem,
            )
            c.start()
            return c

        def ostart(i):
            c = pltpu.make_async_copy(
                rows_v.at[i % 2],
                out_hbm.at[pl.ds(base + i * chunk, chunk)],
                osem,
            )
            c.start()
            return c

        g = [None] * n_chunks
        o = [None] * n_chunks
        g[0] = gstart(0)
        for i in range(n_chunks):
            if i + 1 < n_chunks:
                if i >= 1:
                    o[i - 1].wait()
                g[i + 1] = gstart(i + 1)
            g[i].wait()
            o[i] = ostart(i)
        if n_chunks >= 2:
            o[n_chunks - 2].wait()
        o[n_chunks - 1].wait()

    return k(idx_flat, table)


def kernel(x, table):
    B = x.shape[0] * x.shape[1]
    b_per_w = B // NW
    idx_flat = x.reshape(-1).astype(jnp.int32)
    out = _sc_gather(idx_flat, table, b_per_w, 832)
    return out.reshape(x.shape + (table.shape[1],))
